# Initial kernel scaffold; baseline (speedup 1.0000x reference)
import jax
import jax.numpy as jnp
from jax.experimental import pallas as pl


def kernel(x1, x2, w1, sh1, w2, sh2, w3, sh3, wfc, bfc):
    raise NotImplementedError("write your pallas kernel here")


# speedup vs baseline: 2.7994x; 2.7994x over previous
import jax
import jax.numpy as jnp
from jax.experimental import pallas as pl
from jax.experimental.pallas import tpu as pltpu

L = 8
H1, H2, H3 = 128, 256, 128
K1, K2, K3 = 8, 5, 3
N_CLASS = 4
OUT_W = 8
TB = 512

_L2R = [(max(0, 2 * p - 2), min(L, 2 * p + 4)) for p in range(L // 2)]
_L3R = [(max(0, 2 * p - 1), min(L, 2 * p + 3)) for p in range(L // 2)]


def _build_w1(w1):
    wb = jnp.zeros((L, L * H1), w1.dtype)
    left = (K1 - 1) // 2
    for t in range(L):
        for s in range(L):
            j = s - t + left
            if 0 <= j < K1:
                wb = wb.at[s, t * H1:(t + 1) * H1].set(w1[j, :])
    return wb


def _build_w2(w2, p):
    s0, s1 = _L2R[p]
    left = (K2 - 1) // 2
    slab = jnp.zeros(((s1 - s0) * H1, 2 * H2), w2.dtype)
    for col, t in enumerate((2 * p, 2 * p + 1)):
        for s in range(s0, s1):
            j = s - t + left
            if 0 <= j < K2:
                slab = slab.at[(s - s0) * H1:(s - s0 + 1) * H1,
                               col * H2:(col + 1) * H2].set(w2[j * H1:(j + 1) * H1, :])
    return slab


def _build_w3(w3, p):
    s0, s1 = _L3R[p]
    left = (K3 - 1) // 2
    slab = jnp.zeros(((s1 - s0) * H2, 2 * H3), w3.dtype)
    for col, t in enumerate((2 * p, 2 * p + 1)):
        for s in range(s0, s1):
            j = s - t + left
            if 0 <= j < K3:
                slab = slab.at[(s - s0) * H2:(s - s0 + 1) * H2,
                               col * H3:(col + 1) * H3].set(w3[j * H2:(j + 1) * H2, :])
    return slab


def _fcn_kernel(x1_ref, x2_ref,
                w1_ref, sh1_ref,
                w2a_ref, w2b_ref, w2c_ref, w2d_ref, sh2_ref,
                w3a_ref, w3b_ref, w3c_ref, w3d_ref, sh3_ref,
                wfc_ref, bfc_ref,
                out_ref):
    tb = x1_ref.shape[0]

    xb = jnp.concatenate([x1_ref[...], x2_ref[...]], axis=0).astype(jnp.bfloat16)
    h1 = jnp.maximum(
        jnp.dot(xb, w1_ref[...], preferred_element_type=jnp.float32)
        + sh1_ref[...], 0.0)
    h1b = h1.astype(jnp.bfloat16)

    z2 = []
    for (s0, s1), wr in zip(_L2R, (w2a_ref, w2b_ref, w2c_ref, w2d_ref)):
        z = jnp.dot(h1b[:, s0 * H1:s1 * H1], wr[...],
                    preferred_element_type=jnp.float32)
        z2.append(jnp.maximum(z + sh2_ref[...], 0.0).astype(jnp.bfloat16))
    h2b = jnp.concatenate(z2, axis=1)

    acc = None
    for (s0, s1), wr in zip(_L3R, (w3a_ref, w3b_ref, w3c_ref, w3d_ref)):
        z = jnp.dot(h2b[:, s0 * H2:s1 * H2], wr[...],
                    preferred_element_type=jnp.float32)
        z = jnp.maximum(z + sh3_ref[...], 0.0)
        c = z[:, :H3] + z[:, H3:]
        acc = c if acc is None else acc + c

    feat = (acc[:tb] + acc[tb:]) * (1.0 / (2 * L))
    logits = jnp.dot(feat.astype(jnp.bfloat16), wfc_ref[...],
                     preferred_element_type=jnp.float32) + bfc_ref[...]
    out_ref[...] = logits[:, :OUT_W]


def kernel(x1, x2, w1, sh1, w2, sh2, w3, sh3, wfc, bfc):
    B = x1.shape[0]
    xs1 = x1.reshape(B, L)
    xs2 = x2.reshape(B, L)
    Bp = (B + TB - 1) // TB * TB
    if Bp != B:
        xs1 = jnp.pad(xs1, ((0, Bp - B), (0, 0)))
        xs2 = jnp.pad(xs2, ((0, Bp - B), (0, 0)))
    nt = Bp // TB

    w1b = _build_w1(w1)
    sh1b = jnp.tile(sh1, (1, L))
    w2s = [_build_w2(w2, p) for p in range(L // 2)]
    sh2b = jnp.concatenate([sh2, sh2], axis=1)
    w3s = [_build_w3(w3, p) for p in range(L // 2)]
    sh3b = jnp.concatenate([sh3, sh3], axis=1)

    x_spec = pl.BlockSpec((TB, L), lambda i: (i, 0))

    def full_spec(a):
        return pl.BlockSpec(a.shape, lambda i, _n=a.ndim: (0,) * _n)

    weights = (w1b, sh1b, *w2s, sh2b, *w3s, sh3b, wfc, bfc)

    out = pl.pallas_call(
        _fcn_kernel,
        out_shape=jax.ShapeDtypeStruct((Bp, OUT_W), jnp.float32),
        grid=(nt,),
        in_specs=[x_spec, x_spec] + [full_spec(w) for w in weights],
        out_specs=pl.BlockSpec((TB, OUT_W), lambda i: (i, 0)),
        compiler_params=pltpu.CompilerParams(
            dimension_semantics=("parallel",),
            vmem_limit_bytes=64 * 1024 * 1024),
    )(xs1, xs2, *weights)
    return out[:B, :N_CLASS]


# speedup vs baseline: 2.8646x; 1.0233x over previous
import jax
import jax.numpy as jnp
from jax.experimental import pallas as pl
from jax.experimental.pallas import tpu as pltpu

L = 8
H1, H2, H3 = 128, 256, 128
K1, K2, K3 = 8, 5, 3
N_CLASS = 4
OUT_W = 4
TB = 1024

_L2R = [(max(0, 2 * p - 2), min(L, 2 * p + 4)) for p in range(L // 2)]
_L3R = [(max(0, 2 * p - 1), min(L, 2 * p + 3)) for p in range(L // 2)]


def _build_w1(w1):
    wb = jnp.zeros((L, L * H1), w1.dtype)
    left = (K1 - 1) // 2
    for t in range(L):
        for s in range(L):
            j = s - t + left
            if 0 <= j < K1:
                wb = wb.at[s, t * H1:(t + 1) * H1].set(w1[j, :])
    return wb


def _build_w2(w2, p):
    s0, s1 = _L2R[p]
    left = (K2 - 1) // 2
    slab = jnp.zeros(((s1 - s0) * H1, 2 * H2), w2.dtype)
    for col, t in enumerate((2 * p, 2 * p + 1)):
        for s in range(s0, s1):
            j = s - t + left
            if 0 <= j < K2:
                slab = slab.at[(s - s0) * H1:(s - s0 + 1) * H1,
                               col * H2:(col + 1) * H2].set(w2[j * H1:(j + 1) * H1, :])
    return slab


def _build_w3(w3, p):
    s0, s1 = _L3R[p]
    left = (K3 - 1) // 2
    slab = jnp.zeros(((s1 - s0) * H2, 2 * H3), w3.dtype)
    for col, t in enumerate((2 * p, 2 * p + 1)):
        for s in range(s0, s1):
            j = s - t + left
            if 0 <= j < K3:
                slab = slab.at[(s - s0) * H2:(s - s0 + 1) * H2,
                               col * H3:(col + 1) * H3].set(w3[j * H2:(j + 1) * H2, :])
    return slab


def _fcn_kernel(x1_ref, x2_ref,
                w1_ref, sh1_ref,
                w2a_ref, w2b_ref, w2c_ref, w2d_ref, sh2_ref,
                w3a_ref, w3b_ref, w3c_ref, w3d_ref, sh3_ref,
                wfc_ref, bfc_ref,
                out_ref):
    tb = x1_ref.shape[0]

    xb = jnp.concatenate([x1_ref[...], x2_ref[...]], axis=0).astype(jnp.bfloat16)
    h1 = jnp.maximum(
        jnp.dot(xb, w1_ref[...], preferred_element_type=jnp.float32)
        + sh1_ref[...], 0.0)
    h1b = h1.astype(jnp.bfloat16)

    z2 = []
    for (s0, s1), wr in zip(_L2R, (w2a_ref, w2b_ref, w2c_ref, w2d_ref)):
        z = jnp.dot(h1b[:, s0 * H1:s1 * H1], wr[...],
                    preferred_element_type=jnp.float32)
        z2.append(jnp.maximum(z + sh2_ref[...], 0.0).astype(jnp.bfloat16))
    h2b = jnp.concatenate(z2, axis=1)

    acc = None
    for (s0, s1), wr in zip(_L3R, (w3a_ref, w3b_ref, w3c_ref, w3d_ref)):
        z = jnp.dot(h2b[:, s0 * H2:s1 * H2], wr[...],
                    preferred_element_type=jnp.float32)
        z = jnp.maximum(z + sh3_ref[...], 0.0)
        c = z[:, :H3] + z[:, H3:]
        acc = c if acc is None else acc + c

    feat = (acc[:tb] + acc[tb:]) * (1.0 / (2 * L))
    logits = jnp.dot(feat.astype(jnp.bfloat16), wfc_ref[...],
                     preferred_element_type=jnp.float32) + bfc_ref[...]
    out_ref[...] = logits[:, :OUT_W]


def kernel(x1, x2, w1, sh1, w2, sh2, w3, sh3, wfc, bfc):
    B = x1.shape[0]
    xs1 = x1.reshape(B, L)
    xs2 = x2.reshape(B, L)
    Bp = (B + TB - 1) // TB * TB
    if Bp != B:
        xs1 = jnp.pad(xs1, ((0, Bp - B), (0, 0)))
        xs2 = jnp.pad(xs2, ((0, Bp - B), (0, 0)))
    nt = Bp // TB

    w1b = _build_w1(w1)
    sh1b = jnp.tile(sh1, (1, L))
    w2s = [_build_w2(w2, p) for p in range(L // 2)]
    sh2b = jnp.concatenate([sh2, sh2], axis=1)
    w3s = [_build_w3(w3, p) for p in range(L // 2)]
    sh3b = jnp.concatenate([sh3, sh3], axis=1)

    x_spec = pl.BlockSpec((TB, L), lambda i: (i, 0))

    def full_spec(a):
        return pl.BlockSpec(a.shape, lambda i, _n=a.ndim: (0,) * _n)

    weights = (w1b, sh1b, *w2s, sh2b, *w3s, sh3b, wfc, bfc)

    out = pl.pallas_call(
        _fcn_kernel,
        out_shape=jax.ShapeDtypeStruct((Bp, OUT_W), jnp.float32),
        grid=(nt,),
        in_specs=[x_spec, x_spec] + [full_spec(w) for w in weights],
        out_specs=pl.BlockSpec((TB, OUT_W), lambda i: (i, 0)),
        compiler_params=pltpu.CompilerParams(
            dimension_semantics=("parallel",),
            vmem_limit_bytes=64 * 1024 * 1024),
    )(xs1, xs2, *weights)
    return out[:B, :N_CLASS]


# speedup vs baseline: 2.9060x; 1.0144x over previous
import jax
import jax.numpy as jnp
from jax.experimental import pallas as pl
from jax.experimental.pallas import tpu as pltpu

L = 8
H1, H2, H3 = 128, 256, 128
K1, K2, K3 = 8, 5, 3
N_CLASS = 4
OUT_W = 4
TB = 2048

_L2R = [(max(0, 2 * p - 2), min(L, 2 * p + 4)) for p in range(L // 2)]
_L3R = [(max(0, 2 * p - 1), min(L, 2 * p + 3)) for p in range(L // 2)]


def _build_w1(w1):
    wb = jnp.zeros((L, L * H1), w1.dtype)
    left = (K1 - 1) // 2
    for t in range(L):
        for s in range(L):
            j = s - t + left
            if 0 <= j < K1:
                wb = wb.at[s, t * H1:(t + 1) * H1].set(w1[j, :])
    return wb


def _build_w2(w2, p):
    s0, s1 = _L2R[p]
    left = (K2 - 1) // 2
    slab = jnp.zeros(((s1 - s0) * H1, 2 * H2), w2.dtype)
    for col, t in enumerate((2 * p, 2 * p + 1)):
        for s in range(s0, s1):
            j = s - t + left
            if 0 <= j < K2:
                slab = slab.at[(s - s0) * H1:(s - s0 + 1) * H1,
                               col * H2:(col + 1) * H2].set(w2[j * H1:(j + 1) * H1, :])
    return slab


def _build_w3(w3, p):
    s0, s1 = _L3R[p]
    left = (K3 - 1) // 2
    slab = jnp.zeros(((s1 - s0) * H2, 2 * H3), w3.dtype)
    for col, t in enumerate((2 * p, 2 * p + 1)):
        for s in range(s0, s1):
            j = s - t + left
            if 0 <= j < K3:
                slab = slab.at[(s - s0) * H2:(s - s0 + 1) * H2,
                               col * H3:(col + 1) * H3].set(w3[j * H2:(j + 1) * H2, :])
    return slab


def _fcn_kernel(x1_ref, x2_ref,
                w1_ref, sh1_ref,
                w2a_ref, w2b_ref, w2c_ref, w2d_ref, sh2_ref,
                w3a_ref, w3b_ref, w3c_ref, w3d_ref, sh3_ref,
                wfc_ref, bfc_ref,
                out_ref):
    tb = x1_ref.shape[0]

    xb = jnp.concatenate([x1_ref[...], x2_ref[...]], axis=0).astype(jnp.bfloat16)
    h1 = jnp.maximum(
        jnp.dot(xb, w1_ref[...], preferred_element_type=jnp.float32)
        + sh1_ref[...], 0.0)
    h1b = h1.astype(jnp.bfloat16)

    z2 = []
    for (s0, s1), wr in zip(_L2R, (w2a_ref, w2b_ref, w2c_ref, w2d_ref)):
        z = jnp.dot(h1b[:, s0 * H1:s1 * H1], wr[...],
                    preferred_element_type=jnp.float32)
        z2.append(jnp.maximum(z + sh2_ref[...], 0.0).astype(jnp.bfloat16))
    h2b = jnp.concatenate(z2, axis=1)

    acc = None
    for (s0, s1), wr in zip(_L3R, (w3a_ref, w3b_ref, w3c_ref, w3d_ref)):
        z = jnp.dot(h2b[:, s0 * H2:s1 * H2], wr[...],
                    preferred_element_type=jnp.float32)
        z = jnp.maximum(z + sh3_ref[...], 0.0)
        c = z[:, :H3] + z[:, H3:]
        acc = c if acc is None else acc + c

    feat = (acc[:tb] + acc[tb:]) * (1.0 / (2 * L))
    logits = jnp.dot(feat.astype(jnp.bfloat16), wfc_ref[...],
                     preferred_element_type=jnp.float32) + bfc_ref[...]
    out_ref[...] = logits[:, :OUT_W]


def kernel(x1, x2, w1, sh1, w2, sh2, w3, sh3, wfc, bfc):
    B = x1.shape[0]
    xs1 = x1.reshape(B, L)
    xs2 = x2.reshape(B, L)
    Bp = (B + TB - 1) // TB * TB
    if Bp != B:
        xs1 = jnp.pad(xs1, ((0, Bp - B), (0, 0)))
        xs2 = jnp.pad(xs2, ((0, Bp - B), (0, 0)))
    nt = Bp // TB

    w1b = _build_w1(w1)
    sh1b = jnp.tile(sh1, (1, L))
    w2s = [_build_w2(w2, p) for p in range(L // 2)]
    sh2b = jnp.concatenate([sh2, sh2], axis=1)
    w3s = [_build_w3(w3, p) for p in range(L // 2)]
    sh3b = jnp.concatenate([sh3, sh3], axis=1)

    x_spec = pl.BlockSpec((TB, L), lambda i: (i, 0))

    def full_spec(a):
        return pl.BlockSpec(a.shape, lambda i, _n=a.ndim: (0,) * _n)

    weights = (w1b, sh1b, *w2s, sh2b, *w3s, sh3b, wfc, bfc)

    out = pl.pallas_call(
        _fcn_kernel,
        out_shape=jax.ShapeDtypeStruct((Bp, OUT_W), jnp.float32),
        grid=(nt,),
        in_specs=[x_spec, x_spec] + [full_spec(w) for w in weights],
        out_specs=pl.BlockSpec((TB, OUT_W), lambda i: (i, 0)),
        compiler_params=pltpu.CompilerParams(
            dimension_semantics=("parallel",),
            vmem_limit_bytes=64 * 1024 * 1024),
    )(xs1, xs2, *weights)
    return out[:B, :N_CLASS]


# speedup vs baseline: 2.9381x; 1.0111x over previous
import jax
import jax.numpy as jnp
from jax.experimental import pallas as pl
from jax.experimental.pallas import tpu as pltpu

L = 8
H1, H2, H3 = 128, 256, 128
K1, K2, K3 = 8, 5, 3
N_CLASS = 4
TB = 2048

_L2R = [(max(0, 2 * p - 2), min(L, 2 * p + 4)) for p in range(L // 2)]
_L3R = [(max(0, 2 * p - 1), min(L, 2 * p + 3)) for p in range(L // 2)]


def _banded_rows(w_ref, cin, k, pair, s_range):
    cout = w_ref.shape[1]
    left = (k - 1) // 2
    zero = jnp.zeros((cin, cout), w_ref.dtype)
    rows = []
    for s in range(*s_range):
        cols = []
        for t in (2 * pair, 2 * pair + 1):
            j = s - t + left
            cols.append(w_ref[j * cin:(j + 1) * cin, :] if 0 <= j < k else zero)
        rows.append(jnp.concatenate(cols, axis=1))
    return jnp.concatenate(rows, axis=0)


def _fcn_kernel(x1_ref, x2_ref,
                w1_ref, sh1_ref, w2_ref, sh2_ref, w3_ref, sh3_ref,
                wfc_ref, bfc_ref,
                out_ref,
                w1b_ref, w2a_ref, w2b_ref, w2c_ref, w2d_ref,
                w3a_ref, w3b_ref, w3c_ref, w3d_ref):
    tb = x1_ref.shape[0]
    w2s = (w2a_ref, w2b_ref, w2c_ref, w2d_ref)
    w3s = (w3a_ref, w3b_ref, w3c_ref, w3d_ref)

    @pl.when(pl.program_id(0) == 0)
    def _build():
        left1 = (K1 - 1) // 2
        zrow = jnp.zeros((1, H1), w1_ref.dtype)
        cols = []
        for t in range(L):
            rows = []
            for s in range(L):
                j = s - t + left1
                rows.append(w1_ref[j:j + 1, :] if 0 <= j < K1 else zrow)
            cols.append(jnp.concatenate(rows, axis=0))
        w1b_ref[...] = jnp.concatenate(cols, axis=1)
        for p in range(L // 2):
            w2s[p][...] = _banded_rows(w2_ref, H1, K2, p, _L2R[p])
            w3s[p][...] = _banded_rows(w3_ref, H2, K3, p, _L3R[p])

    sh1b = jnp.tile(sh1_ref[...], (1, L))
    sh2b = jnp.tile(sh2_ref[...], (1, 2))
    sh3b = jnp.tile(sh3_ref[...], (1, 2))

    xb = jnp.concatenate([x1_ref[...], x2_ref[...]], axis=0).astype(jnp.bfloat16)
    h1 = jnp.maximum(
        jnp.dot(xb, w1b_ref[...], preferred_element_type=jnp.float32) + sh1b, 0.0)
    h1b = h1.astype(jnp.bfloat16)

    z2 = []
    for (s0, s1), wr in zip(_L2R, w2s):
        z = jnp.dot(h1b[:, s0 * H1:s1 * H1], wr[...],
                    preferred_element_type=jnp.float32)
        z2.append(jnp.maximum(z + sh2b, 0.0).astype(jnp.bfloat16))
    h2b = jnp.concatenate(z2, axis=1)

    acc = None
    for (s0, s1), wr in zip(_L3R, w3s):
        z = jnp.dot(h2b[:, s0 * H2:s1 * H2], wr[...],
                    preferred_element_type=jnp.float32)
        z = jnp.maximum(z + sh3b, 0.0)
        c = z[:, :H3] + z[:, H3:]
        acc = c if acc is None else acc + c

    feat = (acc[:tb] + acc[tb:]) * (1.0 / (2 * L))
    logits = jnp.dot(feat.astype(jnp.bfloat16), wfc_ref[...],
                     preferred_element_type=jnp.float32) + bfc_ref[...]
    out_ref[...] = logits[:, :N_CLASS]


def kernel(x1, x2, w1, sh1, w2, sh2, w3, sh3, wfc, bfc):
    B = x1.shape[0]
    xs1 = x1.reshape(B, L)
    xs2 = x2.reshape(B, L)
    Bp = (B + TB - 1) // TB * TB
    if Bp != B:
        xs1 = jnp.pad(xs1, ((0, Bp - B), (0, 0)))
        xs2 = jnp.pad(xs2, ((0, Bp - B), (0, 0)))
    nt = Bp // TB

    x_spec = pl.BlockSpec((TB, L), lambda i: (i, 0))

    def full_spec(a):
        return pl.BlockSpec(a.shape, lambda i, _n=a.ndim: (0,) * _n)

    weights = (w1, sh1, w2, sh2, w3, sh3, wfc, bfc)
    scratch = [pltpu.VMEM((L, L * H1), jnp.bfloat16)]
    scratch += [pltpu.VMEM(((s1 - s0) * H1, 2 * H2), jnp.bfloat16)
                for s0, s1 in _L2R]
    scratch += [pltpu.VMEM(((s1 - s0) * H2, 2 * H3), jnp.bfloat16)
                for s0, s1 in _L3R]

    out = pl.pallas_call(
        _fcn_kernel,
        out_shape=jax.ShapeDtypeStruct((Bp, N_CLASS), jnp.float32),
        grid=(nt,),
        in_specs=[x_spec, x_spec] + [full_spec(w) for w in weights],
        out_specs=pl.BlockSpec((TB, N_CLASS), lambda i: (i, 0)),
        scratch_shapes=scratch,
        compiler_params=pltpu.CompilerParams(
            dimension_semantics=("arbitrary",),
            vmem_limit_bytes=64 * 1024 * 1024),
    )(xs1, xs2, *weights)
    return out[:B, :N_CLASS]
